# baseline (speedup 1.0000x reference)
import functools

import jax
import jax.numpy as jnp
from jax import lax
from jax.experimental import pallas as pl
from jax.experimental.pallas import tpu as pltpu
from jax.experimental.pallas import tpu_sc as plsc

N = 10000
E = 320000
D = 128
G = 64

NC = 2
NS = 16
CHUNK = 125
EROWS = E // CHUNK
EROWS_PER_SC = EROWS // NC
EROWS_PER_TILE = EROWS_PER_SC // NS
STAGE = 40
NP = 10240
NROWS_PER_TILE = NP // NS
ZCH = 32
R = 1000
GRID = N // R


def _zero2d(ref, nrows, ncols):
  z = jnp.zeros((16,), jnp.float32)

  def body(i, c):
    for k in range(ncols // 16):
      ref[i, pl.ds(k * 16, 16)] = z
    return c

  lax.fori_loop(0, nrows, body, 0)


def _deg_body(dst_hbm, out_hbm, didx_v, ones_v, zeros_v, sem0, zsem, acc_sh):
  cid = lax.axis_index("c")
  sid = lax.axis_index("s")
  nbase = pl.multiple_of(sid * NROWS_PER_TILE, NROWS_PER_TILE)
  wid = cid * NS + sid

  pltpu.async_copy(dst_hbm.at[wid], didx_v, sem0)

  one = jnp.full((16,), 1.0, jnp.float32)
  zero = jnp.zeros((16,), jnp.float32)

  def init(i, c):
    ones_v[i, :] = one
    return c

  lax.fori_loop(0, CHUNK, init, 0)

  def initz(i, c):
    zeros_v[i, :] = zero
    return c

  lax.fori_loop(0, ZCH, initz, 0)

  for k in range(NROWS_PER_TILE // ZCH):
    pltpu.async_copy(zeros_v, acc_sh.at[pl.ds(nbase + k * ZCH, ZCH)], zsem)
  for k in range(NROWS_PER_TILE // ZCH):
    pltpu.make_async_copy(zeros_v, acc_sh.at[pl.ds(nbase + k * ZCH, ZCH)],
                          zsem).wait()
  pltpu.make_async_copy(dst_hbm.at[wid], didx_v, sem0).wait()
  plsc.subcore_barrier()

  PIPE = 16

  def fire(j, c):
    pltpu.async_copy(ones_v, acc_sh.at[didx_v.at[j]], zsem, add=True)
    return c

  def fire_drain(j, c):
    pltpu.async_copy(ones_v, acc_sh.at[didx_v.at[j]], zsem, add=True)
    pltpu.make_async_copy(ones_v, acc_sh.at[didx_v.at[j - PIPE]], zsem).wait()
    return c

  def drain(j, c):
    pltpu.make_async_copy(ones_v, acc_sh.at[didx_v.at[j]], zsem).wait()
    return c

  lax.fori_loop(0, PIPE, fire, 0)
  lax.fori_loop(PIPE, EROWS_PER_TILE, fire_drain, 0)
  lax.fori_loop(EROWS_PER_TILE - PIPE, EROWS_PER_TILE, drain, 0)
  plsc.subcore_barrier()

  pltpu.sync_copy(
      acc_sh.at[pl.ds(nbase, NROWS_PER_TILE)],
      out_hbm.at[cid, pl.ds(nbase, NROWS_PER_TILE), :],
  )


@functools.cache
def _deg_call():
  return pl.kernel(
      _deg_body,
      out_type=jax.ShapeDtypeStruct((NC, NP, 16), jnp.float32),
      mesh=plsc.VectorSubcoreMesh(core_axis_name="c", subcore_axis_name="s",
                                  num_cores=NC, num_subcores=NS),
      scratch_types=[
          pltpu.VMEM((EROWS_PER_TILE, CHUNK), jnp.int32),
          pltpu.VMEM((CHUNK, 16), jnp.float32),
          pltpu.VMEM((ZCH, 16), jnp.float32),
          pltpu.SemaphoreType.DMA,
          pltpu.SemaphoreType.DMA,
          pltpu.VMEM_SHARED((NP, 16), jnp.float32),
      ],
  )


def _scatter_body(g_hbm, src_hbm, dst_hbm, out_hbm, sidx_v, didx_v,
                  rows0_v, rows1_v, zeros_v, sem0, sem1, ssem0, ssem1, zsem,
                  acc_sh):
  cid = lax.axis_index("c")
  sid = lax.axis_index("s")
  wid = cid * NS + sid
  nbase = pl.multiple_of(sid * NROWS_PER_TILE, NROWS_PER_TILE)

  _zero2d(zeros_v, ZCH, D)
  for k in range(NROWS_PER_TILE // ZCH):
    pltpu.async_copy(zeros_v, acc_sh.at[pl.ds(nbase + k * ZCH, ZCH)], zsem)
  for k in range(NROWS_PER_TILE // ZCH):
    pltpu.make_async_copy(zeros_v, acc_sh.at[pl.ds(nbase + k * ZCH, ZCH)],
                          zsem).wait()
  plsc.subcore_barrier()

  for s in range(EROWS_PER_TILE // STAGE):
    pltpu.async_copy(src_hbm.at[wid, pl.ds(s * STAGE, STAGE)], sidx_v, sem0)
    pltpu.async_copy(dst_hbm.at[wid, pl.ds(s * STAGE, STAGE)], didx_v, sem1)
    pltpu.make_async_copy(src_hbm.at[wid, pl.ds(s * STAGE, STAGE)], sidx_v,
                          sem0).wait()
    pltpu.make_async_copy(dst_hbm.at[wid, pl.ds(s * STAGE, STAGE)], didx_v,
                          sem1).wait()
    pltpu.async_copy(g_hbm.at[sidx_v.at[0]], rows0_v, sem0)

    def body(g, c):
      j0 = 2 * g

      @pl.when(g > 0)
      def _():
        pltpu.make_async_copy(rows1_v, acc_sh.at[didx_v.at[j0 - 1]],
                              ssem1).wait()

      pltpu.async_copy(g_hbm.at[sidx_v.at[j0 + 1]], rows1_v, sem1)
      pltpu.make_async_copy(g_hbm.at[sidx_v.at[j0]], rows0_v, sem0).wait()
      pltpu.async_copy(rows0_v, acc_sh.at[didx_v.at[j0]], ssem0, add=True)
      pltpu.make_async_copy(g_hbm.at[sidx_v.at[j0 + 1]], rows1_v, sem1).wait()
      pltpu.async_copy(rows1_v, acc_sh.at[didx_v.at[j0 + 1]], ssem1, add=True)
      pltpu.make_async_copy(rows0_v, acc_sh.at[didx_v.at[j0]], ssem0).wait()

      @pl.when(g < STAGE // 2 - 1)
      def _():
        pltpu.async_copy(g_hbm.at[sidx_v.at[j0 + 2]], rows0_v, sem0)

      return c

    lax.fori_loop(0, STAGE // 2, body, 0)
    pltpu.make_async_copy(rows1_v, acc_sh.at[didx_v.at[STAGE - 1]],
                          ssem1).wait()
  plsc.subcore_barrier()

  pltpu.sync_copy(
      acc_sh.at[pl.ds(nbase, NROWS_PER_TILE)],
      out_hbm.at[cid, pl.ds(nbase, NROWS_PER_TILE), :],
  )


@functools.cache
def _scatter_call():
  return pl.kernel(
      _scatter_body,
      out_type=jax.ShapeDtypeStruct((NC, NP, D), jnp.float32),
      mesh=plsc.VectorSubcoreMesh(core_axis_name="c", subcore_axis_name="s",
                                  num_cores=NC, num_subcores=NS),
      scratch_types=[
          pltpu.VMEM((STAGE, CHUNK), jnp.int32),
          pltpu.VMEM((STAGE, CHUNK), jnp.int32),
          pltpu.VMEM((CHUNK, D), jnp.float32),
          pltpu.VMEM((CHUNK, D), jnp.float32),
          pltpu.VMEM((ZCH, D), jnp.float32),
          pltpu.SemaphoreType.DMA,
          pltpu.SemaphoreType.DMA,
          pltpu.SemaphoreType.DMA,
          pltpu.SemaphoreType.DMA,
          pltpu.SemaphoreType.DMA,
          pltpu.VMEM_SHARED((NP, D), jnp.float32),
      ],
  )


def _dinv_block(degp_ref):
  deg = degp_ref[0, :, 0:1] + degp_ref[1, :, 0:1] + 1.0
  return lax.rsqrt(deg)


def _tc_first_body(x_ref, degp_ref, w_ref, g_ref):
  dinv = _dinv_block(degp_ref)
  g_ref[...] = jnp.dot(x_ref[...] * dinv, w_ref[...],
                       preferred_element_type=jnp.float32)


def _tc_mid_body(sp_ref, g_ref, degp_ref, b_ref, w_ref, gout_ref):
  dinv = _dinv_block(degp_ref)
  h = jax.nn.relu(dinv * (sp_ref[0] + sp_ref[1] + g_ref[...]) + b_ref[...])
  gout_ref[...] = jnp.dot(h * dinv, w_ref[...],
                          preferred_element_type=jnp.float32)


def _tc_pool_body(sp_ref, g_ref, degp_ref, b_ref, batch_ref, out_ref, cnt_ref):
  i = pl.program_id(0)
  dinv = _dinv_block(degp_ref)
  h = dinv * (sp_ref[0] + sp_ref[1] + g_ref[...]) + b_ref[...]
  ids = batch_ref[0, 0, :]
  onehot = (ids[:, None] == lax.broadcasted_iota(jnp.int32, (R, G), 1))
  p = onehot.astype(jnp.float32)
  contrib = lax.dot_general(p, h, (((0,), (0,)), ((), ())),
                            preferred_element_type=jnp.float32)
  c = jnp.broadcast_to(jnp.sum(p, axis=0)[:, None], (G, D))

  @pl.when(i == 0)
  def _():
    out_ref[...] = contrib
    cnt_ref[...] = c

  @pl.when(i > 0)
  def _():
    out_ref[...] += contrib
    cnt_ref[...] += c

  @pl.when(i == pl.num_programs(0) - 1)
  def _():
    out_ref[...] = out_ref[...] / jnp.maximum(cnt_ref[...], 1.0)


_spec_rows = pl.BlockSpec((R, D), lambda i: (i, 0))
_spec_parts = pl.BlockSpec((NC, R, D), lambda i: (0, i, 0))
_spec_deg = pl.BlockSpec((NC, R, 16), lambda i: (0, i, 0))
_spec_w = pl.BlockSpec((D, D), lambda i: (0, 0))
_spec_b = pl.BlockSpec((1, D), lambda i: (0, 0))

_tc_first = pl.pallas_call(
    _tc_first_body,
    grid=(GRID,),
    in_specs=[_spec_rows, _spec_deg, _spec_w],
    out_specs=_spec_rows,
    out_shape=jax.ShapeDtypeStruct((N, D), jnp.float32),
)

_tc_mid = pl.pallas_call(
    _tc_mid_body,
    grid=(GRID,),
    in_specs=[_spec_parts, _spec_rows, _spec_deg, _spec_b, _spec_w],
    out_specs=_spec_rows,
    out_shape=jax.ShapeDtypeStruct((N, D), jnp.float32),
)

_tc_pool = pl.pallas_call(
    _tc_pool_body,
    grid=(GRID,),
    in_specs=[_spec_parts, _spec_rows, _spec_deg, _spec_b,
              pl.BlockSpec((1, 1, R), lambda i: (i, 0, 0))],
    out_specs=pl.BlockSpec((G, D), lambda i: (0, 0)),
    out_shape=jax.ShapeDtypeStruct((G, D), jnp.float32),
    scratch_shapes=[pltpu.VMEM((G, D), jnp.float32)],
)


def kernel(x, edge_index, batch, W1, b1, W2, b2, W3, b3):
  src2d = edge_index[0].reshape(NC * NS, EROWS_PER_TILE, CHUNK)
  dst2d = edge_index[1].reshape(NC * NS, EROWS_PER_TILE, CHUNK)
  batch3d = batch.reshape(GRID, 1, R)

  degp = _deg_call()(dst2d)
  g1 = _tc_first(x, degp, W1)
  s1 = _scatter_call()(g1, src2d, dst2d)
  g2 = _tc_mid(s1, g1, degp, b1.reshape(1, D), W2)
  s2 = _scatter_call()(g2, src2d, dst2d)
  g3 = _tc_mid(s2, g2, degp, b2.reshape(1, D), W3)
  s3 = _scatter_call()(g3, src2d, dst2d)
  return _tc_pool(s3, g3, degp, b3.reshape(1, D), batch3d)


# speedup vs baseline: 1.2688x; 1.2688x over previous
import functools

import jax
import jax.numpy as jnp
from jax import lax
from jax.experimental import pallas as pl
from jax.experimental.pallas import tpu as pltpu
from jax.experimental.pallas import tpu_sc as plsc

N = 10000
E = 320000
D = 128
G = 64

NC = 2
NS = 16
CHUNK = 125
EROWS = E // CHUNK
EROWS_PER_SC = EROWS // NC
EROWS_PER_TILE = EROWS_PER_SC // NS
STAGE = 40
NP = 10240
NROWS_PER_TILE = NP // NS
ZCH = 32
R = 1000
GRID = N // R


def _zero2d(ref, nrows, ncols):
  z = jnp.zeros((16,), jnp.float32)

  def body(i, c):
    for k in range(ncols // 16):
      ref[i, pl.ds(k * 16, 16)] = z
    return c

  lax.fori_loop(0, nrows, body, 0)


def _deg_body(dst_hbm, out_hbm, didx_v, ones_v, zeros_v, sem0, zsem, acc_sh):
  cid = lax.axis_index("c")
  sid = lax.axis_index("s")
  nbase = pl.multiple_of(sid * NROWS_PER_TILE, NROWS_PER_TILE)
  wid = cid * NS + sid

  pltpu.async_copy(dst_hbm.at[wid], didx_v, sem0)

  one = jnp.full((16,), 1.0, jnp.float32)
  zero = jnp.zeros((16,), jnp.float32)

  def init(i, c):
    ones_v[i, :] = one
    return c

  lax.fori_loop(0, CHUNK, init, 0)

  def initz(i, c):
    zeros_v[i, :] = zero
    return c

  lax.fori_loop(0, ZCH, initz, 0)

  for k in range(NROWS_PER_TILE // ZCH):
    pltpu.async_copy(zeros_v, acc_sh.at[pl.ds(nbase + k * ZCH, ZCH)], zsem)
  for k in range(NROWS_PER_TILE // ZCH):
    pltpu.make_async_copy(zeros_v, acc_sh.at[pl.ds(nbase + k * ZCH, ZCH)],
                          zsem).wait()
  pltpu.make_async_copy(dst_hbm.at[wid], didx_v, sem0).wait()
  plsc.subcore_barrier()

  PIPE = 16

  def fire(j, c):
    pltpu.async_copy(ones_v, acc_sh.at[didx_v.at[j]], zsem, add=True)
    return c

  def fire_drain(j, c):
    pltpu.async_copy(ones_v, acc_sh.at[didx_v.at[j]], zsem, add=True)
    pltpu.make_async_copy(ones_v, acc_sh.at[didx_v.at[j - PIPE]], zsem).wait()
    return c

  def drain(j, c):
    pltpu.make_async_copy(ones_v, acc_sh.at[didx_v.at[j]], zsem).wait()
    return c

  lax.fori_loop(0, PIPE, fire, 0)
  lax.fori_loop(PIPE, EROWS_PER_TILE, fire_drain, 0)
  lax.fori_loop(EROWS_PER_TILE - PIPE, EROWS_PER_TILE, drain, 0)
  plsc.subcore_barrier()

  pltpu.sync_copy(
      acc_sh.at[pl.ds(nbase, NROWS_PER_TILE)],
      out_hbm.at[cid, pl.ds(nbase, NROWS_PER_TILE), :],
  )


@functools.cache
def _deg_call():
  return pl.kernel(
      _deg_body,
      out_type=jax.ShapeDtypeStruct((NC, NP, 16), jnp.float32),
      mesh=plsc.VectorSubcoreMesh(core_axis_name="c", subcore_axis_name="s",
                                  num_cores=NC, num_subcores=NS),
      scratch_types=[
          pltpu.VMEM((EROWS_PER_TILE, CHUNK), jnp.int32),
          pltpu.VMEM((CHUNK, 16), jnp.float32),
          pltpu.VMEM((ZCH, 16), jnp.float32),
          pltpu.SemaphoreType.DMA,
          pltpu.SemaphoreType.DMA,
          pltpu.VMEM_SHARED((NP, 16), jnp.float32),
      ],
  )


def _scatter_body(g_hbm, src_hbm, dst_hbm, out_hbm, sidx_v, didx_v,
                  rows0_v, rows1_v, zeros_v, sem0, sem1, zsem, acc_sh):
  cid = lax.axis_index("c")
  sid = lax.axis_index("s")
  wid = cid * NS + sid
  nbase = pl.multiple_of(sid * NROWS_PER_TILE, NROWS_PER_TILE)

  pltpu.async_copy(src_hbm.at[wid, pl.ds(0, STAGE)], sidx_v, sem0)
  pltpu.async_copy(dst_hbm.at[wid, pl.ds(0, STAGE)], didx_v, sem1)
  _zero2d(zeros_v, ZCH, D)
  for k in range(NROWS_PER_TILE // ZCH):
    pltpu.async_copy(zeros_v, acc_sh.at[pl.ds(nbase + k * ZCH, ZCH)], zsem)
  pltpu.make_async_copy(src_hbm.at[wid, pl.ds(0, STAGE)], sidx_v, sem0).wait()
  pltpu.make_async_copy(dst_hbm.at[wid, pl.ds(0, STAGE)], didx_v, sem1).wait()
  pltpu.async_copy(g_hbm.at[sidx_v.at[0]], rows0_v, sem0)
  for k in range(NROWS_PER_TILE // ZCH):
    pltpu.make_async_copy(zeros_v, acc_sh.at[pl.ds(nbase + k * ZCH, ZCH)],
                          zsem).wait()
  plsc.subcore_barrier()

  for s in range(EROWS_PER_TILE // STAGE):
    if s > 0:
      pltpu.sync_copy(src_hbm.at[wid, pl.ds(s * STAGE, STAGE)], sidx_v)
      pltpu.sync_copy(dst_hbm.at[wid, pl.ds(s * STAGE, STAGE)], didx_v)
      pltpu.async_copy(g_hbm.at[sidx_v.at[0]], rows0_v, sem0)

    def body(g, c):
      j0 = 2 * g
      pltpu.async_copy(g_hbm.at[sidx_v.at[j0 + 1]], rows1_v, sem1)
      pltpu.make_async_copy(g_hbm.at[sidx_v.at[j0]], rows0_v, sem0).wait()
      pltpu.sync_copy(rows0_v, acc_sh.at[didx_v.at[j0]], add=True)

      @pl.when(g < STAGE // 2 - 1)
      def _():
        pltpu.async_copy(g_hbm.at[sidx_v.at[j0 + 2]], rows0_v, sem0)

      pltpu.make_async_copy(g_hbm.at[sidx_v.at[j0 + 1]], rows1_v, sem1).wait()
      pltpu.sync_copy(rows1_v, acc_sh.at[didx_v.at[j0 + 1]], add=True)
      return c

    lax.fori_loop(0, STAGE // 2, body, 0)
  plsc.subcore_barrier()

  pltpu.sync_copy(
      acc_sh.at[pl.ds(nbase, NROWS_PER_TILE)],
      out_hbm.at[cid, pl.ds(nbase, NROWS_PER_TILE), :],
  )


@functools.cache
def _scatter_call():
  return pl.kernel(
      _scatter_body,
      out_type=jax.ShapeDtypeStruct((NC, NP, D), jnp.float32),
      mesh=plsc.VectorSubcoreMesh(core_axis_name="c", subcore_axis_name="s",
                                  num_cores=NC, num_subcores=NS),
      scratch_types=[
          pltpu.VMEM((STAGE, CHUNK), jnp.int32),
          pltpu.VMEM((STAGE, CHUNK), jnp.int32),
          pltpu.VMEM((CHUNK, D), jnp.float32),
          pltpu.VMEM((CHUNK, D), jnp.float32),
          pltpu.VMEM((ZCH, D), jnp.float32),
          pltpu.SemaphoreType.DMA,
          pltpu.SemaphoreType.DMA,
          pltpu.SemaphoreType.DMA,
          pltpu.VMEM_SHARED((NP, D), jnp.float32),
      ],
  )


def _dinv_block(degp_ref):
  deg = degp_ref[0, :, 0:1] + degp_ref[1, :, 0:1] + 1.0
  return lax.rsqrt(deg)


def _tc_first_body(x_ref, degp_ref, w_ref, g_ref):
  dinv = _dinv_block(degp_ref)
  g_ref[...] = jnp.dot(x_ref[...] * dinv, w_ref[...],
                       preferred_element_type=jnp.float32)


def _tc_mid_body(sp_ref, g_ref, degp_ref, b_ref, w_ref, gout_ref):
  dinv = _dinv_block(degp_ref)
  h = jax.nn.relu(dinv * (sp_ref[0] + sp_ref[1] + g_ref[...]) + b_ref[...])
  gout_ref[...] = jnp.dot(h * dinv, w_ref[...],
                          preferred_element_type=jnp.float32)


def _tc_pool_body(sp_ref, g_ref, degp_ref, b_ref, batch_ref, out_ref, cnt_ref):
  i = pl.program_id(0)
  dinv = _dinv_block(degp_ref)
  h = dinv * (sp_ref[0] + sp_ref[1] + g_ref[...]) + b_ref[...]
  ids = batch_ref[0, 0, :]
  onehot = (ids[:, None] == lax.broadcasted_iota(jnp.int32, (R, G), 1))
  p = onehot.astype(jnp.float32)
  contrib = lax.dot_general(p, h, (((0,), (0,)), ((), ())),
                            preferred_element_type=jnp.float32)
  c = jnp.broadcast_to(jnp.sum(p, axis=0)[:, None], (G, D))

  @pl.when(i == 0)
  def _():
    out_ref[...] = contrib
    cnt_ref[...] = c

  @pl.when(i > 0)
  def _():
    out_ref[...] += contrib
    cnt_ref[...] += c

  @pl.when(i == pl.num_programs(0) - 1)
  def _():
    out_ref[...] = out_ref[...] / jnp.maximum(cnt_ref[...], 1.0)


_spec_rows = pl.BlockSpec((R, D), lambda i: (i, 0))
_spec_parts = pl.BlockSpec((NC, R, D), lambda i: (0, i, 0))
_spec_deg = pl.BlockSpec((NC, R, 16), lambda i: (0, i, 0))
_spec_w = pl.BlockSpec((D, D), lambda i: (0, 0))
_spec_b = pl.BlockSpec((1, D), lambda i: (0, 0))

_tc_first = pl.pallas_call(
    _tc_first_body,
    grid=(GRID,),
    in_specs=[_spec_rows, _spec_deg, _spec_w],
    out_specs=_spec_rows,
    out_shape=jax.ShapeDtypeStruct((N, D), jnp.float32),
)

_tc_mid = pl.pallas_call(
    _tc_mid_body,
    grid=(GRID,),
    in_specs=[_spec_parts, _spec_rows, _spec_deg, _spec_b, _spec_w],
    out_specs=_spec_rows,
    out_shape=jax.ShapeDtypeStruct((N, D), jnp.float32),
)

_tc_pool = pl.pallas_call(
    _tc_pool_body,
    grid=(GRID,),
    in_specs=[_spec_parts, _spec_rows, _spec_deg, _spec_b,
              pl.BlockSpec((1, 1, R), lambda i: (i, 0, 0))],
    out_specs=pl.BlockSpec((G, D), lambda i: (0, 0)),
    out_shape=jax.ShapeDtypeStruct((G, D), jnp.float32),
    scratch_shapes=[pltpu.VMEM((G, D), jnp.float32)],
)


def kernel(x, edge_index, batch, W1, b1, W2, b2, W3, b3):
  src2d = edge_index[0].reshape(NC * NS, EROWS_PER_TILE, CHUNK)
  dst2d = edge_index[1].reshape(NC * NS, EROWS_PER_TILE, CHUNK)
  batch3d = batch.reshape(GRID, 1, R)

  degp = _deg_call()(dst2d)
  g1 = _tc_first(x, degp, W1)
  s1 = _scatter_call()(g1, src2d, dst2d)
  g2 = _tc_mid(s1, g1, degp, b1.reshape(1, D), W2)
  s2 = _scatter_call()(g2, src2d, dst2d)
  g3 = _tc_mid(s2, g2, degp, b2.reshape(1, D), W3)
  s3 = _scatter_call()(g3, src2d, dst2d)
  return _tc_pool(s3, g3, degp, b3.reshape(1, D), batch3d)


# speedup vs baseline: 1.2980x; 1.0230x over previous
import functools

import jax
import jax.numpy as jnp
from jax import lax
from jax.experimental import pallas as pl
from jax.experimental.pallas import tpu as pltpu
from jax.experimental.pallas import tpu_sc as plsc

N = 10000
E = 320000
D = 128
G = 64

NC = 2
NS = 16
CHUNK = 125
EROWS = E // CHUNK
EROWS_PER_SC = EROWS // NC
EROWS_PER_TILE = EROWS_PER_SC // NS
STAGE = 40
NP = 10240
NROWS_PER_TILE = NP // NS
ZCH = 32
R = 2000
GRID = N // R


def _zero2d(ref, nrows, ncols):
  z = jnp.zeros((16,), jnp.float32)

  def body(i, c):
    for k in range(ncols // 16):
      ref[i, pl.ds(k * 16, 16)] = z
    return c

  lax.fori_loop(0, nrows, body, 0)


def _deg_body(dst_hbm, out_hbm, didx_v, ones_v, zeros_v, sem0, zsem, acc_sh):
  cid = lax.axis_index("c")
  sid = lax.axis_index("s")
  nbase = pl.multiple_of(sid * NROWS_PER_TILE, NROWS_PER_TILE)
  wid = cid * NS + sid

  pltpu.async_copy(dst_hbm.at[wid], didx_v, sem0)

  one = jnp.full((16,), 1.0, jnp.float32)
  zero = jnp.zeros((16,), jnp.float32)

  def init(i, c):
    ones_v[i, :] = one
    return c

  lax.fori_loop(0, CHUNK, init, 0)

  def initz(i, c):
    zeros_v[i, :] = zero
    return c

  lax.fori_loop(0, ZCH, initz, 0)

  for k in range(NROWS_PER_TILE // ZCH):
    pltpu.async_copy(zeros_v, acc_sh.at[pl.ds(nbase + k * ZCH, ZCH)], zsem)
  for k in range(NROWS_PER_TILE // ZCH):
    pltpu.make_async_copy(zeros_v, acc_sh.at[pl.ds(nbase + k * ZCH, ZCH)],
                          zsem).wait()
  pltpu.make_async_copy(dst_hbm.at[wid], didx_v, sem0).wait()
  plsc.subcore_barrier()

  PIPE = 16

  def fire(j, c):
    pltpu.async_copy(ones_v, acc_sh.at[didx_v.at[j]], zsem, add=True)
    return c

  def fire_drain(j, c):
    pltpu.async_copy(ones_v, acc_sh.at[didx_v.at[j]], zsem, add=True)
    pltpu.make_async_copy(ones_v, acc_sh.at[didx_v.at[j - PIPE]], zsem).wait()
    return c

  def drain(j, c):
    pltpu.make_async_copy(ones_v, acc_sh.at[didx_v.at[j]], zsem).wait()
    return c

  lax.fori_loop(0, PIPE, fire, 0)
  lax.fori_loop(PIPE, EROWS_PER_TILE, fire_drain, 0)
  lax.fori_loop(EROWS_PER_TILE - PIPE, EROWS_PER_TILE, drain, 0)
  plsc.subcore_barrier()

  pltpu.sync_copy(
      acc_sh.at[pl.ds(nbase, NROWS_PER_TILE)],
      out_hbm.at[cid, pl.ds(nbase, NROWS_PER_TILE), :],
  )


@functools.cache
def _deg_call():
  return pl.kernel(
      _deg_body,
      out_type=jax.ShapeDtypeStruct((NC, NP, 16), jnp.float32),
      mesh=plsc.VectorSubcoreMesh(core_axis_name="c", subcore_axis_name="s",
                                  num_cores=NC, num_subcores=NS),
      scratch_types=[
          pltpu.VMEM((EROWS_PER_TILE, CHUNK), jnp.int32),
          pltpu.VMEM((CHUNK, 16), jnp.float32),
          pltpu.VMEM((ZCH, 16), jnp.float32),
          pltpu.SemaphoreType.DMA,
          pltpu.SemaphoreType.DMA,
          pltpu.VMEM_SHARED((NP, 16), jnp.float32),
      ],
  )


def _scatter_body(g_hbm, src_hbm, dst_hbm, out_hbm, sidx_v, didx_v,
                  rows0_v, rows1_v, zeros_v, sem0, sem1, zsem, acc_sh):
  cid = lax.axis_index("c")
  sid = lax.axis_index("s")
  wid = cid * NS + sid
  nbase = pl.multiple_of(sid * NROWS_PER_TILE, NROWS_PER_TILE)

  pltpu.async_copy(src_hbm.at[wid, pl.ds(0, STAGE)], sidx_v, sem0)
  pltpu.async_copy(dst_hbm.at[wid, pl.ds(0, STAGE)], didx_v, sem1)
  _zero2d(zeros_v, ZCH, D)
  for k in range(NROWS_PER_TILE // ZCH):
    pltpu.async_copy(zeros_v, acc_sh.at[pl.ds(nbase + k * ZCH, ZCH)], zsem)
  pltpu.make_async_copy(src_hbm.at[wid, pl.ds(0, STAGE)], sidx_v, sem0).wait()
  pltpu.make_async_copy(dst_hbm.at[wid, pl.ds(0, STAGE)], didx_v, sem1).wait()
  pltpu.async_copy(g_hbm.at[sidx_v.at[0]], rows0_v, sem0)
  for k in range(NROWS_PER_TILE // ZCH):
    pltpu.make_async_copy(zeros_v, acc_sh.at[pl.ds(nbase + k * ZCH, ZCH)],
                          zsem).wait()
  plsc.subcore_barrier()

  for s in range(EROWS_PER_TILE // STAGE):
    if s > 0:
      pltpu.sync_copy(src_hbm.at[wid, pl.ds(s * STAGE, STAGE)], sidx_v)
      pltpu.sync_copy(dst_hbm.at[wid, pl.ds(s * STAGE, STAGE)], didx_v)
      pltpu.async_copy(g_hbm.at[sidx_v.at[0]], rows0_v, sem0)

    def body(g, c):
      j0 = 2 * g
      pltpu.async_copy(g_hbm.at[sidx_v.at[j0 + 1]], rows1_v, sem1)
      pltpu.make_async_copy(g_hbm.at[sidx_v.at[j0]], rows0_v, sem0).wait()
      pltpu.sync_copy(rows0_v, acc_sh.at[didx_v.at[j0]], add=True)

      @pl.when(g < STAGE // 2 - 1)
      def _():
        pltpu.async_copy(g_hbm.at[sidx_v.at[j0 + 2]], rows0_v, sem0)

      pltpu.make_async_copy(g_hbm.at[sidx_v.at[j0 + 1]], rows1_v, sem1).wait()
      pltpu.sync_copy(rows1_v, acc_sh.at[didx_v.at[j0 + 1]], add=True)
      return c

    lax.fori_loop(0, STAGE // 2, body, 0)
  plsc.subcore_barrier()

  pltpu.sync_copy(
      acc_sh.at[pl.ds(nbase, NROWS_PER_TILE)],
      out_hbm.at[cid, pl.ds(nbase, NROWS_PER_TILE), :],
  )


@functools.cache
def _scatter_call():
  return pl.kernel(
      _scatter_body,
      out_type=jax.ShapeDtypeStruct((NC, NP, D), jnp.float32),
      mesh=plsc.VectorSubcoreMesh(core_axis_name="c", subcore_axis_name="s",
                                  num_cores=NC, num_subcores=NS),
      scratch_types=[
          pltpu.VMEM((STAGE, CHUNK), jnp.int32),
          pltpu.VMEM((STAGE, CHUNK), jnp.int32),
          pltpu.VMEM((CHUNK, D), jnp.float32),
          pltpu.VMEM((CHUNK, D), jnp.float32),
          pltpu.VMEM((ZCH, D), jnp.float32),
          pltpu.SemaphoreType.DMA,
          pltpu.SemaphoreType.DMA,
          pltpu.SemaphoreType.DMA,
          pltpu.VMEM_SHARED((NP, D), jnp.float32),
      ],
  )


def _dinv_block(degp_ref):
  deg = degp_ref[0, :, 0:1] + degp_ref[1, :, 0:1] + 1.0
  return lax.rsqrt(deg)


def _tc_first_body(x_ref, degp_ref, w_ref, g_ref):
  dinv = _dinv_block(degp_ref)
  g_ref[...] = jnp.dot(x_ref[...] * dinv, w_ref[...],
                       preferred_element_type=jnp.float32)


def _tc_mid_body(sp_ref, g_ref, degp_ref, b_ref, w_ref, gout_ref):
  dinv = _dinv_block(degp_ref)
  h = jax.nn.relu(dinv * (sp_ref[0] + sp_ref[1] + g_ref[...]) + b_ref[...])
  gout_ref[...] = jnp.dot(h * dinv, w_ref[...],
                          preferred_element_type=jnp.float32)


def _tc_pool_body(sp_ref, g_ref, degp_ref, b_ref, batch_ref, out_ref, cnt_ref):
  i = pl.program_id(0)
  dinv = _dinv_block(degp_ref)
  h = dinv * (sp_ref[0] + sp_ref[1] + g_ref[...]) + b_ref[...]
  ids = batch_ref[0, 0, :]
  onehot = (ids[:, None] == lax.broadcasted_iota(jnp.int32, (R, G), 1))
  p = onehot.astype(jnp.float32)
  contrib = lax.dot_general(p, h, (((0,), (0,)), ((), ())),
                            preferred_element_type=jnp.float32)
  c = jnp.broadcast_to(jnp.sum(p, axis=0)[:, None], (G, D))

  @pl.when(i == 0)
  def _():
    out_ref[...] = contrib
    cnt_ref[...] = c

  @pl.when(i > 0)
  def _():
    out_ref[...] += contrib
    cnt_ref[...] += c

  @pl.when(i == pl.num_programs(0) - 1)
  def _():
    out_ref[...] = out_ref[...] / jnp.maximum(cnt_ref[...], 1.0)


_spec_rows = pl.BlockSpec((R, D), lambda i: (i, 0))
_spec_parts = pl.BlockSpec((NC, R, D), lambda i: (0, i, 0))
_spec_deg = pl.BlockSpec((NC, R, 16), lambda i: (0, i, 0))
_spec_w = pl.BlockSpec((D, D), lambda i: (0, 0))
_spec_b = pl.BlockSpec((1, D), lambda i: (0, 0))

_tc_first = pl.pallas_call(
    _tc_first_body,
    grid=(GRID,),
    in_specs=[_spec_rows, _spec_deg, _spec_w],
    out_specs=_spec_rows,
    out_shape=jax.ShapeDtypeStruct((N, D), jnp.float32),
)

_tc_mid = pl.pallas_call(
    _tc_mid_body,
    grid=(GRID,),
    in_specs=[_spec_parts, _spec_rows, _spec_deg, _spec_b, _spec_w],
    out_specs=_spec_rows,
    out_shape=jax.ShapeDtypeStruct((N, D), jnp.float32),
)

_tc_pool = pl.pallas_call(
    _tc_pool_body,
    grid=(GRID,),
    in_specs=[_spec_parts, _spec_rows, _spec_deg, _spec_b,
              pl.BlockSpec((1, 1, R), lambda i: (i, 0, 0))],
    out_specs=pl.BlockSpec((G, D), lambda i: (0, 0)),
    out_shape=jax.ShapeDtypeStruct((G, D), jnp.float32),
    scratch_shapes=[pltpu.VMEM((G, D), jnp.float32)],
)


def kernel(x, edge_index, batch, W1, b1, W2, b2, W3, b3):
  src2d = edge_index[0].reshape(NC * NS, EROWS_PER_TILE, CHUNK)
  dst2d = edge_index[1].reshape(NC * NS, EROWS_PER_TILE, CHUNK)
  batch3d = batch.reshape(GRID, 1, R)

  degp = _deg_call()(dst2d)
  g1 = _tc_first(x, degp, W1)
  s1 = _scatter_call()(g1, src2d, dst2d)
  g2 = _tc_mid(s1, g1, degp, b1.reshape(1, D), W2)
  s2 = _scatter_call()(g2, src2d, dst2d)
  g3 = _tc_mid(s2, g2, degp, b2.reshape(1, D), W3)
  s3 = _scatter_call()(g3, src2d, dst2d)
  return _tc_pool(s3, g3, degp, b3.reshape(1, D), batch3d)
